# baseline (speedup 1.0000x reference)
import jax
import jax.numpy as jnp
from jax.experimental import pallas as pl
from jax.experimental.pallas import tpu as pltpu

_BS, _N, _D = 16, 16384, 128
_BLK = 2048
_NB = _N // _BLK


def _body(mask_ref, pred_ref, target_ref, out_ref, acc_ref):
    b = pl.program_id(0)
    i = pl.program_id(1)

    @pl.when(jnp.logical_and(b == 0, i == 0))
    def _():
        out_ref[0, 0] = 0.0

    @pl.when(i == 0)
    def _():
        acc_ref[0] = 0.0
        acc_ref[1] = 0.0
        acc_ref[2] = 0.0
        acc_ref[3] = 0.0

    m = (mask_ref[0, 0, :] != 0).astype(jnp.float32)
    mf = m[:, None]
    p = pred_ref[0]
    t = target_ref[0]
    mp = p * mf
    mt = t * mf
    acc_ref[0] += jnp.sum(mp * t)
    acc_ref[1] += jnp.sum(mp * p)
    acc_ref[2] += jnp.sum(mt * t)
    acc_ref[3] += jnp.sum(m)

    @pl.when(i == _NB - 1)
    def _():
        dot = acc_ref[0]
        pp = acc_ref[1]
        tt = acc_ref[2]
        cnt = acc_ref[3]
        denom = jnp.sqrt(pp) * jnp.sqrt(tt)
        safe = jnp.where(denom > 0.0, denom, 1.0)
        loss = jnp.where(cnt > 0.0, -dot / safe, 0.0)
        out_ref[0, 0] += loss / _BS


def kernel(pred, target, mask):
    mask3 = mask.reshape(_BS * _NB, 1, _BLK)
    out = pl.pallas_call(
        _body,
        grid=(_BS, _NB),
        in_specs=[
            pl.BlockSpec((1, 1, _BLK), lambda b, i: (b * _NB + i, 0, 0)),
            pl.BlockSpec((1, _BLK, _D), lambda b, i: (b, i, 0)),
            pl.BlockSpec((1, _BLK, _D), lambda b, i: (b, i, 0)),
        ],
        out_specs=pl.BlockSpec(memory_space=pltpu.SMEM),
        out_shape=jax.ShapeDtypeStruct((1, 1), jnp.float32),
        scratch_shapes=[pltpu.SMEM((4,), jnp.float32)],
    )(mask3, pred, target)
    return out[0, 0]


# speedup vs baseline: 1.3768x; 1.3768x over previous
import jax
import jax.numpy as jnp
from jax.experimental import pallas as pl
from jax.experimental.pallas import tpu as pltpu

_BS, _N, _D = 16, 16384, 128
_BLK = 4096
_NB = _N // _BLK


def _body(mask_ref, pred_ref, target_ref, out_ref, acc_ref, cnt_ref):
    b = pl.program_id(0)
    i = pl.program_id(1)

    @pl.when(jnp.logical_and(b == 0, i == 0))
    def _():
        out_ref[0, 0] = 0.0

    @pl.when(i == 0)
    def _():
        acc_ref[...] = jnp.zeros_like(acc_ref)
        cnt_ref[0] = 0.0

    m = (mask_ref[0, 0, :] != 0).astype(jnp.float32)
    mf = m[:, None]
    p = pred_ref[0]
    t = target_ref[0]
    mp = (p * mf).reshape(_BLK // 8, 8, _D)
    mt = (t * mf).reshape(_BLK // 8, 8, _D)
    pr = p.reshape(_BLK // 8, 8, _D)
    tr = t.reshape(_BLK // 8, 8, _D)
    acc_ref[0] += jnp.sum(mp * tr, axis=0)
    acc_ref[1] += jnp.sum(mp * pr, axis=0)
    acc_ref[2] += jnp.sum(mt * tr, axis=0)
    cnt_ref[0] += jnp.sum(m)

    @pl.when(i == _NB - 1)
    def _():
        dot = jnp.sum(acc_ref[0])
        pp = jnp.sum(acc_ref[1])
        tt = jnp.sum(acc_ref[2])
        cnt = cnt_ref[0]
        denom = jnp.sqrt(pp) * jnp.sqrt(tt)
        safe = jnp.where(denom > 0.0, denom, 1.0)
        loss = jnp.where(cnt > 0.0, -dot / safe, 0.0)
        out_ref[0, 0] += loss / _BS


def kernel(pred, target, mask):
    mask3 = mask.reshape(_BS * _NB, 1, _BLK)
    out = pl.pallas_call(
        _body,
        grid=(_BS, _NB),
        in_specs=[
            pl.BlockSpec((1, 1, _BLK), lambda b, i: (b * _NB + i, 0, 0)),
            pl.BlockSpec((1, _BLK, _D), lambda b, i: (b, i, 0)),
            pl.BlockSpec((1, _BLK, _D), lambda b, i: (b, i, 0)),
        ],
        out_specs=pl.BlockSpec(memory_space=pltpu.SMEM),
        out_shape=jax.ShapeDtypeStruct((1, 1), jnp.float32),
        scratch_shapes=[pltpu.VMEM((3, 8, _D), jnp.float32),
                        pltpu.SMEM((1,), jnp.float32)],
    )(mask3, pred, target)
    return out[0, 0]


# speedup vs baseline: 1.6603x; 1.2059x over previous
import jax
import jax.numpy as jnp
from jax.experimental import pallas as pl
from jax.experimental.pallas import tpu as pltpu

_BS, _N, _D = 16, 16384, 128
_BLK = 8192
_NB = _N // _BLK


def _body(mask_ref, pred_ref, target_ref, out_ref, acc_ref, cnt_ref):
    b = pl.program_id(0)
    i = pl.program_id(1)

    @pl.when(jnp.logical_and(b == 0, i == 0))
    def _():
        out_ref[0, 0] = 0.0

    @pl.when(i == 0)
    def _():
        acc_ref[...] = jnp.zeros_like(acc_ref)
        cnt_ref[0] = 0.0

    m = (mask_ref[0, 0, :] != 0).astype(jnp.float32)
    mf = m[:, None]
    p = pred_ref[0]
    t = target_ref[0]
    mp = (p * mf).reshape(_BLK // 8, 8, _D)
    mt = (t * mf).reshape(_BLK // 8, 8, _D)
    pr = p.reshape(_BLK // 8, 8, _D)
    tr = t.reshape(_BLK // 8, 8, _D)
    acc_ref[0] += jnp.sum(mp * tr, axis=0)
    acc_ref[1] += jnp.sum(mp * pr, axis=0)
    acc_ref[2] += jnp.sum(mt * tr, axis=0)
    cnt_ref[0] += jnp.sum(m)

    @pl.when(i == _NB - 1)
    def _():
        dot = jnp.sum(acc_ref[0])
        pp = jnp.sum(acc_ref[1])
        tt = jnp.sum(acc_ref[2])
        cnt = cnt_ref[0]
        denom = jnp.sqrt(pp) * jnp.sqrt(tt)
        safe = jnp.where(denom > 0.0, denom, 1.0)
        loss = jnp.where(cnt > 0.0, -dot / safe, 0.0)
        out_ref[0, 0] += loss / _BS


def kernel(pred, target, mask):
    mask3 = mask.reshape(_BS * _NB, 1, _BLK)
    out = pl.pallas_call(
        _body,
        grid=(_BS, _NB),
        in_specs=[
            pl.BlockSpec((1, 1, _BLK), lambda b, i: (b * _NB + i, 0, 0)),
            pl.BlockSpec((1, _BLK, _D), lambda b, i: (b, i, 0)),
            pl.BlockSpec((1, _BLK, _D), lambda b, i: (b, i, 0)),
        ],
        out_specs=pl.BlockSpec(memory_space=pltpu.SMEM),
        out_shape=jax.ShapeDtypeStruct((1, 1), jnp.float32),
        scratch_shapes=[pltpu.VMEM((3, 8, _D), jnp.float32),
                        pltpu.SMEM((1,), jnp.float32)],
    )(mask3, pred, target)
    return out[0, 0]


# speedup vs baseline: 1.7283x; 1.0410x over previous
import jax
import jax.numpy as jnp
from jax.experimental import pallas as pl
from jax.experimental.pallas import tpu as pltpu

_BS, _N, _D = 16, 16384, 128
_BLK = 16384
_NB = _N // _BLK


def _body(mask_ref, pred_ref, target_ref, out_ref, acc_ref, cnt_ref):
    b = pl.program_id(0)
    i = pl.program_id(1)

    @pl.when(jnp.logical_and(b == 0, i == 0))
    def _():
        out_ref[0, 0] = 0.0

    @pl.when(i == 0)
    def _():
        acc_ref[...] = jnp.zeros_like(acc_ref)
        cnt_ref[0] = 0.0

    m = (mask_ref[0, 0, :] != 0).astype(jnp.float32)
    mf = m[:, None]
    p = pred_ref[0]
    t = target_ref[0]
    mp = (p * mf).reshape(_BLK // 8, 8, _D)
    mt = (t * mf).reshape(_BLK // 8, 8, _D)
    pr = p.reshape(_BLK // 8, 8, _D)
    tr = t.reshape(_BLK // 8, 8, _D)
    acc_ref[0] += jnp.sum(mp * tr, axis=0)
    acc_ref[1] += jnp.sum(mp * pr, axis=0)
    acc_ref[2] += jnp.sum(mt * tr, axis=0)
    cnt_ref[0] += jnp.sum(m)

    @pl.when(i == _NB - 1)
    def _():
        dot = jnp.sum(acc_ref[0])
        pp = jnp.sum(acc_ref[1])
        tt = jnp.sum(acc_ref[2])
        cnt = cnt_ref[0]
        denom = jnp.sqrt(pp) * jnp.sqrt(tt)
        safe = jnp.where(denom > 0.0, denom, 1.0)
        loss = jnp.where(cnt > 0.0, -dot / safe, 0.0)
        out_ref[0, 0] += loss / _BS


def kernel(pred, target, mask):
    mask3 = mask.reshape(_BS * _NB, 1, _BLK)
    out = pl.pallas_call(
        _body,
        grid=(_BS, _NB),
        in_specs=[
            pl.BlockSpec((1, 1, _BLK), lambda b, i: (b * _NB + i, 0, 0)),
            pl.BlockSpec((1, _BLK, _D), lambda b, i: (b, i, 0)),
            pl.BlockSpec((1, _BLK, _D), lambda b, i: (b, i, 0)),
        ],
        out_specs=pl.BlockSpec(memory_space=pltpu.SMEM),
        out_shape=jax.ShapeDtypeStruct((1, 1), jnp.float32),
        scratch_shapes=[pltpu.VMEM((3, 8, _D), jnp.float32),
                        pltpu.SMEM((1,), jnp.float32)],
    )(mask3, pred, target)
    return out[0, 0]
